# baseline (speedup 1.0000x reference)
import functools

import jax
import jax.numpy as jnp
from jax import lax
from jax.experimental import pallas as pl
from jax.experimental.pallas import tpu as pltpu
from jax.experimental.pallas import tpu_sc as plsc

_N = 50000
_E = 800000
_H = 64
_G = 128
_HH = 32

_EP = 802816
_EROWS = _EP // 128
_SUB_ROWS = _EROWS // 16
_W_ROWS = _EROWS // 32
_ACC_ROWS = 51200
_STRIPE = _ACC_ROWS // 16
_LAST_FLUSH = _N - 15 * _STRIPE

_R = 512
_NBLK = 98
_PR = 128
_PBLK = 391

_f32 = jnp.float32

_mesh = plsc.VectorSubcoreMesh(core_axis_name="c", subcore_axis_name="s")
_sc_params = pltpu.CompilerParams(use_tc_tiling_on_sc=False)


def _dot(a, b):
    return lax.dot_general(a, b, (((1,), (0,)), ((), ())),
                           preferred_element_type=_f32)


def _flush(acc, out, s):
    @pl.when(s < 15)
    def _():
        pltpu.sync_copy(acc.at[pl.ds(s * _STRIPE, _STRIPE)],
                        out.at[pl.ds(s * _STRIPE, _STRIPE)])

    @pl.when(s == 15)
    def _():
        pltpu.sync_copy(acc.at[pl.ds(15 * _STRIPE, _LAST_FLUSH)],
                        out.at[pl.ds(15 * _STRIPE, _LAST_FLUSH)])


def _deg_kernel(dst2, deg_a, deg_b, dbuf, obuf, zbuf, acc):
    c = lax.axis_index("c")
    s = lax.axis_index("s")

    @pl.loop(0, 128)
    def _fill(r):
        obuf[r, pl.ds(0, 16)] = jnp.full((16,), 1.0, _f32)
        zbuf[r, pl.ds(0, 16)] = jnp.zeros((16,), _f32)

    @pl.loop(0, _STRIPE, step=128)
    def _zero(r0):
        pltpu.sync_copy(zbuf, acc.at[pl.ds(s * _STRIPE + r0, 128)])

    plsc.subcore_barrier()

    w = c * 16 + s
    @pl.loop(0, _W_ROWS, step=4)
    def _chunk(k):
        r0 = w * _W_ROWS + k
        pltpu.sync_copy(dst2.at[pl.ds(r0, 4)], dbuf)
        for j in range(4):
            pltpu.sync_copy(obuf, acc.at[dbuf.at[j]], add=True)

    plsc.subcore_barrier()

    for cid, out in ((0, deg_a), (1, deg_b)):
        @pl.when(c == cid)
        def _():
            _flush(acc, out, s)


@jax.jit
def _sc_degree(dst2):
    out = jax.ShapeDtypeStruct((_N, 16), _f32)
    kern = pl.kernel(
        _deg_kernel,
        out_type=[out, out],
        mesh=_mesh,
        compiler_params=_sc_params,
        scratch_types=[
            pltpu.VMEM((4, 128), jnp.int32),
            pltpu.VMEM((128, 16), _f32),
            pltpu.VMEM((128, 16), _f32),
            pltpu.VMEM_SHARED((_ACC_ROWS, 16), _f32),
        ],
    )
    return kern(dst2)


_TROWS = 50048
_TSTRIPE = _TROWS // 16
_TLAST = _N - 15 * _TSTRIPE


def _agg_kernel(src2, dst2, hq0, hq1, hq2, hq3, oq0, oq1, oq2, oq3,
                sbuf0, dbuf0, gbuf0, sbuf1, dbuf1, gbuf1, tbl, acc, sem):
    c = lax.axis_index("c")
    s = lax.axis_index("s")
    base = s * _SUB_ROWS
    n_chunks = _SUB_ROWS // 2

    @pl.loop(0, 256)
    def _fill(r):
        gbuf0[r, pl.ds(0, 16)] = jnp.zeros((16,), _f32)

    def stage_and_zero(hq):
        @pl.when(s < 15)
        def _():
            pltpu.sync_copy(hq.at[pl.ds(s * _TSTRIPE, _TSTRIPE)],
                            tbl.at[pl.ds(s * _TSTRIPE, _TSTRIPE)])

        @pl.when(s == 15)
        def _():
            pltpu.sync_copy(hq.at[pl.ds(15 * _TSTRIPE, _TLAST)],
                            tbl.at[pl.ds(15 * _TSTRIPE, _TLAST)])

        for i in range(12):
            pltpu.sync_copy(gbuf0, acc.at[pl.ds(s * _STRIPE + i * 256, 256)])
        pltpu.sync_copy(gbuf0.at[pl.ds(0, 128)],
                        acc.at[pl.ds(s * _STRIPE + 3072, 128)])

    def edge_sweep():
        def fire(chunk, sb, db, gb):
            r0 = base + chunk * 2
            pltpu.sync_copy(src2.at[pl.ds(r0, 2)], sb)
            pltpu.sync_copy(dst2.at[pl.ds(r0, 2)], db)
            for j in range(2):
                pltpu.async_copy(tbl.at[sb.at[j]],
                                 gb.at[pl.ds(j * 128, 128)], sem)

        def drain_scatter(sb, db, gb):
            for j in range(2):
                pltpu.make_async_copy(tbl.at[sb.at[j]],
                                      gb.at[pl.ds(j * 128, 128)], sem).wait()
            for j in range(2):
                pltpu.sync_copy(gb.at[pl.ds(j * 128, 128)],
                                acc.at[db.at[j]], add=True)

        fire(0, sbuf0, dbuf0, gbuf0)

        @pl.loop(0, n_chunks - 2, step=2)
        def _pair(k):
            fire(k + 1, sbuf1, dbuf1, gbuf1)
            drain_scatter(sbuf0, dbuf0, gbuf0)
            fire(k + 2, sbuf0, dbuf0, gbuf0)
            drain_scatter(sbuf1, dbuf1, gbuf1)

        fire(n_chunks - 1, sbuf1, dbuf1, gbuf1)
        drain_scatter(sbuf0, dbuf0, gbuf0)
        drain_scatter(sbuf1, dbuf1, gbuf1)

    quarters = ((0, (hq0, oq0), (hq1, oq1)), (1, (hq2, oq2), (hq3, oq3)))
    for p in range(2):
        for cid, qa, qb in quarters:
            hq, oq = (qa, qb)[p]

            @pl.when(c == cid)
            def _():
                stage_and_zero(hq)

        plsc.subcore_barrier()
        edge_sweep()
        plsc.subcore_barrier()

        for cid, qa, qb in quarters:
            hq, oq = (qa, qb)[p]

            @pl.when(c == cid)
            def _():
                _flush(acc, oq, s)

        if p == 0:
            plsc.subcore_barrier()

            @pl.loop(0, 256)
            def _refill(r):
                gbuf0[r, pl.ds(0, 16)] = jnp.zeros((16,), _f32)


@jax.jit
def _sc_aggregate(src2, dst2, hq0, hq1, hq2, hq3):
    out = jax.ShapeDtypeStruct((_N, 16), _f32)
    kern = pl.kernel(
        _agg_kernel,
        out_type=[out, out, out, out],
        mesh=_mesh,
        compiler_params=_sc_params,
        scratch_types=[
            pltpu.VMEM((2, 128), jnp.int32),
            pltpu.VMEM((2, 128), jnp.int32),
            pltpu.VMEM((256, 16), _f32),
            pltpu.VMEM((2, 128), jnp.int32),
            pltpu.VMEM((2, 128), jnp.int32),
            pltpu.VMEM((256, 16), _f32),
            pltpu.VMEM_SHARED((_TROWS, 16), _f32),
            pltpu.VMEM_SHARED((_ACC_ROWS, 16), _f32),
            pltpu.SemaphoreType.DMA,
        ],
    )
    return kern(src2, dst2, hq0, hq1, hq2, hq3)


def _quarter_out_specs():
    return [pl.BlockSpec((_R, 16), lambda i: (i, 0)) for _ in range(4)]


def _quarter_out_shapes():
    return [jax.ShapeDtypeStruct((_N, 16), _f32) for _ in range(4)]


def _store_quarters(hw, refs):
    for q, ref in enumerate(refs):
        ref[...] = hw[:, q * 16:(q + 1) * 16]


def _t1_kernel(dega_ref, degb_ref, x_ref, w_ref, dinv_ref, *q_refs):
    deg = dega_ref[:, 0:1] + degb_ref[:, 0:1] + 1.0
    dinv = 1.0 / jnp.sqrt(deg)
    dinv_ref[...] = jnp.broadcast_to(dinv, (_R, 8))
    hw = _dot(x_ref[...], w_ref[...]) * dinv
    _store_quarters(hw, q_refs)


@jax.jit
def _tc_first(deg_a, deg_b, x8, w1):
    return pl.pallas_call(
        _t1_kernel,
        grid=(_NBLK,),
        in_specs=[
            pl.BlockSpec((_R, 16), lambda i: (i, 0)),
            pl.BlockSpec((_R, 16), lambda i: (i, 0)),
            pl.BlockSpec((_R, 8), lambda i: (i, 0)),
            pl.BlockSpec((8, _H), lambda i: (0, 0)),
        ],
        out_specs=[pl.BlockSpec((_R, 8), lambda i: (i, 0))]
        + _quarter_out_specs(),
        out_shape=[jax.ShapeDtypeStruct((_N, 8), _f32)]
        + _quarter_out_shapes(),
    )(deg_a, deg_b, x8, w1)


def _stats_kernel(a0, a1, a2, a3, h0, h1, h2, h3, dinv_ref, b_ref,
                  z_ref, st_ref, sacc):
    i = pl.program_id(0)

    @pl.when(i == 0)
    def _():
        sacc[...] = jnp.zeros((8, _H), _f32)

    agg = jnp.concatenate([a0[...], a1[...], a2[...], a3[...]], axis=1)
    hws = jnp.concatenate([h0[...], h1[...], h2[...], h3[...]], axis=1)
    z = dinv_ref[:, 0:1] * (agg + hws) + b_ref[...]
    valid = (i * _R + lax.broadcasted_iota(jnp.int32, (_R, 1), 0)) < _N
    zm = jnp.where(valid, z, 0.0)
    sacc[0:1, :] += jnp.sum(zm, axis=0, keepdims=True)
    sacc[1:2, :] += jnp.sum(zm * zm, axis=0, keepdims=True)
    z_ref[...] = z

    @pl.when(i == _NBLK - 1)
    def _():
        st_ref[...] = sacc[...]


@jax.jit
def _tc_stats(aggq, hwsq, dinv8, bvec):
    return pl.pallas_call(
        _stats_kernel,
        grid=(_NBLK,),
        in_specs=[pl.BlockSpec((_R, 16), lambda i: (i, 0))
                  for _ in range(8)]
        + [
            pl.BlockSpec((_R, 8), lambda i: (i, 0)),
            pl.BlockSpec((1, _H), lambda i: (0, 0)),
        ],
        out_specs=[
            pl.BlockSpec((_R, _H), lambda i: (i, 0)),
            pl.BlockSpec((8, _H), lambda i: (0, 0)),
        ],
        out_shape=[
            jax.ShapeDtypeStruct((_N, _H), _f32),
            jax.ShapeDtypeStruct((8, _H), _f32),
        ],
        scratch_shapes=[pltpu.VMEM((8, _H), _f32)],
    )(*aggq, *hwsq, dinv8, bvec)


def _bn_relu(z, st, g, be):
    mean = st[0:1, :] * (1.0 / _N)
    var = st[1:2, :] * (1.0 / _N) - mean * mean
    return jnp.maximum((z - mean) / jnp.sqrt(var + 1e-5) * g + be, 0.0)


def _apply_kernel(z_ref, st_ref, g_ref, be_ref, w_ref, dinv_ref, *q_refs):
    h = _bn_relu(z_ref[...], st_ref[...], g_ref[...], be_ref[...])
    hw = _dot(h, w_ref[...]) * dinv_ref[:, 0:1]
    _store_quarters(hw, q_refs)


@jax.jit
def _tc_apply(z, st, gvec, bevec, wn, dinv8):
    return pl.pallas_call(
        _apply_kernel,
        grid=(_NBLK,),
        in_specs=[
            pl.BlockSpec((_R, _H), lambda i: (i, 0)),
            pl.BlockSpec((8, _H), lambda i: (0, 0)),
            pl.BlockSpec((1, _H), lambda i: (0, 0)),
            pl.BlockSpec((1, _H), lambda i: (0, 0)),
            pl.BlockSpec((_H, _H), lambda i: (0, 0)),
            pl.BlockSpec((_R, 8), lambda i: (i, 0)),
        ],
        out_specs=_quarter_out_specs(),
        out_shape=_quarter_out_shapes(),
    )(z, st, gvec, bevec, wn, dinv8)


def _final_kernel(z_ref, st_ref, g_ref, be_ref, h_ref):
    h_ref[...] = _bn_relu(z_ref[...], st_ref[...], g_ref[...], be_ref[...])


@jax.jit
def _tc_final(z, st, gvec, bevec):
    return pl.pallas_call(
        _final_kernel,
        grid=(_NBLK,),
        in_specs=[
            pl.BlockSpec((_R, _H), lambda i: (i, 0)),
            pl.BlockSpec((8, _H), lambda i: (0, 0)),
            pl.BlockSpec((1, _H), lambda i: (0, 0)),
            pl.BlockSpec((1, _H), lambda i: (0, 0)),
        ],
        out_specs=pl.BlockSpec((_R, _H), lambda i: (i, 0)),
        out_shape=jax.ShapeDtypeStruct((_N, _H), _f32),
    )(z, st, gvec, bevec)


def _pool_kernel(h_ref, b_ref, sum_ref, max_ref, cnt_ref, ssum, smax, scnt):
    i = pl.program_id(0)

    @pl.when(i == 0)
    def _():
        ssum[...] = jnp.zeros((_G, _H), _f32)
        smax[...] = jnp.full((_G, _H), -jnp.inf, _f32)
        scnt[...] = jnp.zeros((_G, 8), _f32)

    h = h_ref[...]
    bb = b_ref[...]
    valid = (i * _PR + lax.broadcasted_iota(jnp.int32, (_PR, 1), 0)) < _N
    g_lo = jnp.min(jnp.where(valid, bb, _G - 1))
    g_hi = jnp.max(jnp.where(valid, bb, 0))

    def body(g, _):
        m = jnp.logical_and(valid, bb == g)
        s = jnp.sum(jnp.where(m, h, 0.0), axis=0, keepdims=True)
        mx = jnp.max(jnp.where(m, h, -jnp.inf), axis=0, keepdims=True)
        cn = jnp.sum(jnp.where(m, 1.0, 0.0))
        ssum[pl.ds(g, 1), :] += s
        smax[pl.ds(g, 1), :] = jnp.maximum(smax[pl.ds(g, 1), :], mx)
        scnt[pl.ds(g, 1), :] += jnp.full((1, 8), cn, _f32)
        return 0

    lax.fori_loop(g_lo, g_hi + 1, body, 0)

    @pl.when(i == _PBLK - 1)
    def _():
        sum_ref[...] = ssum[...]
        max_ref[...] = smax[...]
        cnt_ref[...] = scnt[...]


@jax.jit
def _tc_pool(h3, batch2):
    return pl.pallas_call(
        _pool_kernel,
        grid=(_PBLK,),
        in_specs=[
            pl.BlockSpec((_PR, _H), lambda i: (i, 0)),
            pl.BlockSpec((_PR, 1), lambda i: (i, 0)),
        ],
        out_specs=[
            pl.BlockSpec((_G, _H), lambda i: (0, 0)),
            pl.BlockSpec((_G, _H), lambda i: (0, 0)),
            pl.BlockSpec((_G, 8), lambda i: (0, 0)),
        ],
        out_shape=[
            jax.ShapeDtypeStruct((_G, _H), _f32),
            jax.ShapeDtypeStruct((_G, _H), _f32),
            jax.ShapeDtypeStruct((_G, 8), _f32),
        ],
        scratch_shapes=[
            pltpu.VMEM((_G, _H), _f32),
            pltpu.VMEM((_G, _H), _f32),
            pltpu.VMEM((_G, 8), _f32),
        ],
    )(h3, batch2)


def _head_kernel(sum_ref, max_ref, cnt_ref, w1_ref, b1_ref, g1_ref, be1_ref,
                 w2_ref, b2_ref, g2_ref, be2_ref, w3_ref, b3_ref, out_ref):
    cnt = jnp.maximum(cnt_ref[:, 0:1], 1.0)
    z = jnp.concatenate([sum_ref[...] / cnt, max_ref[...]], axis=1)

    def bn_exact(a, g, be):
        m = jnp.mean(a, axis=0, keepdims=True)
        v = jnp.mean((a - m) * (a - m), axis=0, keepdims=True)
        return jnp.maximum((a - m) / jnp.sqrt(v + 1e-5) * g + be, 0.0)

    a1 = bn_exact(_dot(z, w1_ref[...]) + b1_ref[...], g1_ref[...], be1_ref[...])
    a2 = bn_exact(_dot(a1, w2_ref[...]) + b2_ref[...], g2_ref[...], be2_ref[...])
    out_ref[...] = _dot(a2, w3_ref[...]) + b3_ref[...]


@jax.jit
def _tc_head(sums, maxs, cnt, p):
    return pl.pallas_call(
        _head_kernel,
        out_shape=jax.ShapeDtypeStruct((_G, 8), _f32),
    )(sums, maxs, cnt,
      p["lin1_W"], p["lin1_b"].reshape(1, _H),
      p["gf1"].reshape(1, _H), p["bf1"].reshape(1, _H),
      p["lin2_W"], p["lin2_b"].reshape(1, _HH),
      p["gf2"].reshape(1, _HH), p["bf2"].reshape(1, _HH),
      jnp.pad(p["lin3_W"], ((0, 0), (0, 6))),
      jnp.pad(p["lin3_b"], (0, 6)).reshape(1, 8))


def kernel(x, edge_index, batch, params):
    pad = _EP - _E
    idx = jnp.arange(pad, dtype=jnp.int32)
    src2 = jnp.concatenate(
        [edge_index[0].astype(jnp.int32), idx * 17 % _N]).reshape(_EROWS, 128)
    dst2 = jnp.concatenate(
        [edge_index[1].astype(jnp.int32),
         _N + idx % (_ACC_ROWS - _N)]).reshape(_EROWS, 128)
    x8 = jnp.pad(x, ((0, 0), (0, 5)))
    w1 = jnp.pad(params["W1"], ((0, 5), (0, 0)))
    batch2 = batch.astype(jnp.int32).reshape(_N, 1)

    deg_a, deg_b = _sc_degree(dst2)
    dinv8, *hwsq = _tc_first(deg_a, deg_b, x8, w1)

    for l, wn in (("1", "W2"), ("2", "W3"), ("3", None)):
        aggq = _sc_aggregate(src2, dst2, *hwsq)
        z, st = _tc_stats(aggq, hwsq, dinv8,
                          params["b" + l].reshape(1, _H))
        gvec = params["g" + l].reshape(1, _H)
        bevec = params["be" + l].reshape(1, _H)
        if wn is not None:
            hwsq = _tc_apply(z, st, gvec, bevec, params[wn], dinv8)
        else:
            h3 = _tc_final(z, st, gvec, bevec)

    sums, maxs, cnt = _tc_pool(h3, batch2)
    out = _tc_head(sums, maxs, cnt, params)
    return out[:, :2]


# speedup vs baseline: 1.4894x; 1.4894x over previous
import functools

import jax
import jax.numpy as jnp
from jax import lax
from jax.experimental import pallas as pl
from jax.experimental.pallas import tpu as pltpu
from jax.experimental.pallas import tpu_sc as plsc

_N = 50000
_E = 800000
_H = 64
_G = 128
_HH = 32

_EP = 802816
_EROWS = _EP // 128
_SUB_ROWS = _EROWS // 16
_W_ROWS = _EROWS // 32
_ACC_ROWS = 51200
_STRIPE = _ACC_ROWS // 16
_LAST_FLUSH = _N - 15 * _STRIPE

_R = 512
_NBLK = 98
_PR = 128
_PBLK = 391

_f32 = jnp.float32

_mesh = plsc.VectorSubcoreMesh(core_axis_name="c", subcore_axis_name="s")
_sc_params = pltpu.CompilerParams(use_tc_tiling_on_sc=False)


def _dot(a, b):
    return lax.dot_general(a, b, (((1,), (0,)), ((), ())),
                           preferred_element_type=_f32)


def _flush(acc, out, s):
    @pl.when(s < 15)
    def _():
        pltpu.sync_copy(acc.at[pl.ds(s * _STRIPE, _STRIPE)],
                        out.at[pl.ds(s * _STRIPE, _STRIPE)])

    @pl.when(s == 15)
    def _():
        pltpu.sync_copy(acc.at[pl.ds(15 * _STRIPE, _LAST_FLUSH)],
                        out.at[pl.ds(15 * _STRIPE, _LAST_FLUSH)])


def _deg_kernel(dst2, deg_a, deg_b, dbuf, obuf, zbuf, acc):
    c = lax.axis_index("c")
    s = lax.axis_index("s")

    @pl.loop(0, 128)
    def _fill(r):
        obuf[r, pl.ds(0, 16)] = jnp.full((16,), 1.0, _f32)
        zbuf[r, pl.ds(0, 16)] = jnp.zeros((16,), _f32)

    @pl.loop(0, _STRIPE, step=128)
    def _zero(r0):
        pltpu.sync_copy(zbuf, acc.at[pl.ds(s * _STRIPE + r0, 128)])

    plsc.subcore_barrier()

    w = c * 16 + s
    @pl.loop(0, _W_ROWS, step=4)
    def _chunk(k):
        r0 = w * _W_ROWS + k
        pltpu.sync_copy(dst2.at[pl.ds(r0, 4)], dbuf)
        for j in range(4):
            pltpu.sync_copy(obuf, acc.at[dbuf.at[j]], add=True)

    plsc.subcore_barrier()

    for cid, out in ((0, deg_a), (1, deg_b)):
        @pl.when(c == cid)
        def _():
            _flush(acc, out, s)


@jax.jit
def _sc_degree(dst2):
    out = jax.ShapeDtypeStruct((_N, 16), _f32)
    kern = pl.kernel(
        _deg_kernel,
        out_type=[out, out],
        mesh=_mesh,
        compiler_params=_sc_params,
        scratch_types=[
            pltpu.VMEM((4, 128), jnp.int32),
            pltpu.VMEM((128, 16), _f32),
            pltpu.VMEM((128, 16), _f32),
            pltpu.VMEM_SHARED((_ACC_ROWS, 16), _f32),
        ],
    )
    return kern(dst2)


def _agg_kernel(src2, dst2, hws_lo, hws_hi, out_lo, out_hi,
                sbuf0, dbuf0, gbuf0, sbuf1, dbuf1, gbuf1, acc, sem):
    c = lax.axis_index("c")
    s = lax.axis_index("s")

    @pl.loop(0, 256)
    def _fill(r):
        gbuf0[r, pl.ds(0, 16)] = jnp.zeros((16,), _f32)
        gbuf0[r, pl.ds(16, 16)] = jnp.zeros((16,), _f32)

    for i in range(12):
        pltpu.sync_copy(gbuf0, acc.at[pl.ds(s * _STRIPE + i * 256, 256)])
    pltpu.sync_copy(gbuf0.at[pl.ds(0, 128)],
                    acc.at[pl.ds(s * _STRIPE + 3072, 128)])

    plsc.subcore_barrier()

    base = s * _SUB_ROWS
    n_chunks = _SUB_ROWS // 2

    def run_core(tbl):
        def fire(chunk, sb, db, gb):
            r0 = base + chunk * 2
            pltpu.sync_copy(src2.at[pl.ds(r0, 2)], sb)
            pltpu.sync_copy(dst2.at[pl.ds(r0, 2)], db)
            for j in range(2):
                pltpu.async_copy(tbl.at[sb.at[j]],
                                 gb.at[pl.ds(j * 128, 128)], sem)

        def drain_scatter(sb, db, gb):
            for j in range(2):
                pltpu.make_async_copy(tbl.at[sb.at[j]],
                                      gb.at[pl.ds(j * 128, 128)], sem).wait()
            for j in range(2):
                pltpu.sync_copy(gb.at[pl.ds(j * 128, 128)],
                                acc.at[db.at[j]], add=True)

        fire(0, sbuf0, dbuf0, gbuf0)

        @pl.loop(0, n_chunks - 2, step=2)
        def _pair(k):
            fire(k + 1, sbuf1, dbuf1, gbuf1)
            drain_scatter(sbuf0, dbuf0, gbuf0)
            fire(k + 2, sbuf0, dbuf0, gbuf0)
            drain_scatter(sbuf1, dbuf1, gbuf1)

        fire(n_chunks - 1, sbuf1, dbuf1, gbuf1)
        drain_scatter(sbuf0, dbuf0, gbuf0)
        drain_scatter(sbuf1, dbuf1, gbuf1)

    for cid, tbl in ((0, hws_lo), (1, hws_hi)):
        @pl.when(c == cid)
        def _():
            run_core(tbl)

    plsc.subcore_barrier()

    for cid, out in ((0, out_lo), (1, out_hi)):
        @pl.when(c == cid)
        def _():
            _flush(acc, out, s)


@jax.jit
def _sc_aggregate(src2, dst2, hws_lo, hws_hi):
    out = jax.ShapeDtypeStruct((_N, _HH), _f32)
    kern = pl.kernel(
        _agg_kernel,
        out_type=[out, out],
        mesh=_mesh,
        compiler_params=_sc_params,
        scratch_types=[
            pltpu.VMEM((2, 128), jnp.int32),
            pltpu.VMEM((2, 128), jnp.int32),
            pltpu.VMEM((256, _HH), _f32),
            pltpu.VMEM((2, 128), jnp.int32),
            pltpu.VMEM((2, 128), jnp.int32),
            pltpu.VMEM((256, _HH), _f32),
            pltpu.VMEM_SHARED((_ACC_ROWS, _HH), _f32),
            pltpu.SemaphoreType.DMA,
        ],
    )
    return kern(src2, dst2, hws_lo, hws_hi)


def _t1_kernel(dega_ref, degb_ref, x_ref, w_ref, dinv_ref, lo_ref, hi_ref):
    deg = dega_ref[:, 0:1] + degb_ref[:, 0:1] + 1.0
    dinv = 1.0 / jnp.sqrt(deg)
    dinv_ref[...] = jnp.broadcast_to(dinv, (_R, 8))
    hw = _dot(x_ref[...], w_ref[...]) * dinv
    lo_ref[...] = hw[:, :_HH]
    hi_ref[...] = hw[:, _HH:]


@jax.jit
def _tc_first(deg_a, deg_b, x8, w1):
    return pl.pallas_call(
        _t1_kernel,
        grid=(_NBLK,),
        in_specs=[
            pl.BlockSpec((_R, 16), lambda i: (i, 0)),
            pl.BlockSpec((_R, 16), lambda i: (i, 0)),
            pl.BlockSpec((_R, 8), lambda i: (i, 0)),
            pl.BlockSpec((8, _H), lambda i: (0, 0)),
        ],
        out_specs=[
            pl.BlockSpec((_R, 8), lambda i: (i, 0)),
            pl.BlockSpec((_R, _HH), lambda i: (i, 0)),
            pl.BlockSpec((_R, _HH), lambda i: (i, 0)),
        ],
        out_shape=[
            jax.ShapeDtypeStruct((_N, 8), _f32),
            jax.ShapeDtypeStruct((_N, _HH), _f32),
            jax.ShapeDtypeStruct((_N, _HH), _f32),
        ],
    )(deg_a, deg_b, x8, w1)


def _stats_kernel(alo_ref, ahi_ref, hlo_ref, hhi_ref, dinv_ref, b_ref,
                  z_ref, st_ref, sacc):
    i = pl.program_id(0)

    @pl.when(i == 0)
    def _():
        sacc[...] = jnp.zeros((8, _H), _f32)

    agg = jnp.concatenate([alo_ref[...], ahi_ref[...]], axis=1)
    hws = jnp.concatenate([hlo_ref[...], hhi_ref[...]], axis=1)
    z = dinv_ref[:, 0:1] * (agg + hws) + b_ref[...]
    valid = (i * _R + lax.broadcasted_iota(jnp.int32, (_R, 1), 0)) < _N
    zm = jnp.where(valid, z, 0.0)
    sacc[0:1, :] += jnp.sum(zm, axis=0, keepdims=True)
    sacc[1:2, :] += jnp.sum(zm * zm, axis=0, keepdims=True)
    z_ref[...] = z

    @pl.when(i == _NBLK - 1)
    def _():
        st_ref[...] = sacc[...]


@jax.jit
def _tc_stats(agg_lo, agg_hi, hws_lo, hws_hi, dinv8, bvec):
    return pl.pallas_call(
        _stats_kernel,
        grid=(_NBLK,),
        in_specs=[
            pl.BlockSpec((_R, _HH), lambda i: (i, 0)),
            pl.BlockSpec((_R, _HH), lambda i: (i, 0)),
            pl.BlockSpec((_R, _HH), lambda i: (i, 0)),
            pl.BlockSpec((_R, _HH), lambda i: (i, 0)),
            pl.BlockSpec((_R, 8), lambda i: (i, 0)),
            pl.BlockSpec((1, _H), lambda i: (0, 0)),
        ],
        out_specs=[
            pl.BlockSpec((_R, _H), lambda i: (i, 0)),
            pl.BlockSpec((8, _H), lambda i: (0, 0)),
        ],
        out_shape=[
            jax.ShapeDtypeStruct((_N, _H), _f32),
            jax.ShapeDtypeStruct((8, _H), _f32),
        ],
        scratch_shapes=[pltpu.VMEM((8, _H), _f32)],
    )(agg_lo, agg_hi, hws_lo, hws_hi, dinv8, bvec)


def _bn_relu(z, st, g, be):
    mean = st[0:1, :] * (1.0 / _N)
    var = st[1:2, :] * (1.0 / _N) - mean * mean
    return jnp.maximum((z - mean) / jnp.sqrt(var + 1e-5) * g + be, 0.0)


def _apply_kernel(z_ref, st_ref, g_ref, be_ref, w_ref, dinv_ref,
                  lo_ref, hi_ref):
    h = _bn_relu(z_ref[...], st_ref[...], g_ref[...], be_ref[...])
    hw = _dot(h, w_ref[...]) * dinv_ref[:, 0:1]
    lo_ref[...] = hw[:, :_HH]
    hi_ref[...] = hw[:, _HH:]


@jax.jit
def _tc_apply(z, st, gvec, bevec, wn, dinv8):
    return pl.pallas_call(
        _apply_kernel,
        grid=(_NBLK,),
        in_specs=[
            pl.BlockSpec((_R, _H), lambda i: (i, 0)),
            pl.BlockSpec((8, _H), lambda i: (0, 0)),
            pl.BlockSpec((1, _H), lambda i: (0, 0)),
            pl.BlockSpec((1, _H), lambda i: (0, 0)),
            pl.BlockSpec((_H, _H), lambda i: (0, 0)),
            pl.BlockSpec((_R, 8), lambda i: (i, 0)),
        ],
        out_specs=[
            pl.BlockSpec((_R, _HH), lambda i: (i, 0)),
            pl.BlockSpec((_R, _HH), lambda i: (i, 0)),
        ],
        out_shape=[
            jax.ShapeDtypeStruct((_N, _HH), _f32),
            jax.ShapeDtypeStruct((_N, _HH), _f32),
        ],
    )(z, st, gvec, bevec, wn, dinv8)


def _final_kernel(z_ref, st_ref, g_ref, be_ref, h_ref):
    h_ref[...] = _bn_relu(z_ref[...], st_ref[...], g_ref[...], be_ref[...])


@jax.jit
def _tc_final(z, st, gvec, bevec):
    return pl.pallas_call(
        _final_kernel,
        grid=(_NBLK,),
        in_specs=[
            pl.BlockSpec((_R, _H), lambda i: (i, 0)),
            pl.BlockSpec((8, _H), lambda i: (0, 0)),
            pl.BlockSpec((1, _H), lambda i: (0, 0)),
            pl.BlockSpec((1, _H), lambda i: (0, 0)),
        ],
        out_specs=pl.BlockSpec((_R, _H), lambda i: (i, 0)),
        out_shape=jax.ShapeDtypeStruct((_N, _H), _f32),
    )(z, st, gvec, bevec)


def _pool_kernel(h_ref, b_ref, sum_ref, max_ref, cnt_ref, ssum, smax, scnt):
    i = pl.program_id(0)

    @pl.when(i == 0)
    def _():
        ssum[...] = jnp.zeros((_G, _H), _f32)
        smax[...] = jnp.full((_G, _H), -jnp.inf, _f32)
        scnt[...] = jnp.zeros((_G, 8), _f32)

    h = h_ref[...]
    bb = b_ref[...]
    valid = (i * _PR + lax.broadcasted_iota(jnp.int32, (_PR, 1), 0)) < _N
    g_lo = jnp.min(jnp.where(valid, bb, _G - 1))
    g_hi = jnp.max(jnp.where(valid, bb, 0))

    def body(g, _):
        m = jnp.logical_and(valid, bb == g)
        s = jnp.sum(jnp.where(m, h, 0.0), axis=0, keepdims=True)
        mx = jnp.max(jnp.where(m, h, -jnp.inf), axis=0, keepdims=True)
        cn = jnp.sum(jnp.where(m, 1.0, 0.0))
        ssum[pl.ds(g, 1), :] += s
        smax[pl.ds(g, 1), :] = jnp.maximum(smax[pl.ds(g, 1), :], mx)
        scnt[pl.ds(g, 1), :] += jnp.full((1, 8), cn, _f32)
        return 0

    lax.fori_loop(g_lo, g_hi + 1, body, 0)

    @pl.when(i == _PBLK - 1)
    def _():
        sum_ref[...] = ssum[...]
        max_ref[...] = smax[...]
        cnt_ref[...] = scnt[...]


@jax.jit
def _tc_pool(h3, batch2):
    return pl.pallas_call(
        _pool_kernel,
        grid=(_PBLK,),
        in_specs=[
            pl.BlockSpec((_PR, _H), lambda i: (i, 0)),
            pl.BlockSpec((_PR, 1), lambda i: (i, 0)),
        ],
        out_specs=[
            pl.BlockSpec((_G, _H), lambda i: (0, 0)),
            pl.BlockSpec((_G, _H), lambda i: (0, 0)),
            pl.BlockSpec((_G, 8), lambda i: (0, 0)),
        ],
        out_shape=[
            jax.ShapeDtypeStruct((_G, _H), _f32),
            jax.ShapeDtypeStruct((_G, _H), _f32),
            jax.ShapeDtypeStruct((_G, 8), _f32),
        ],
        scratch_shapes=[
            pltpu.VMEM((_G, _H), _f32),
            pltpu.VMEM((_G, _H), _f32),
            pltpu.VMEM((_G, 8), _f32),
        ],
    )(h3, batch2)


def _head_kernel(sum_ref, max_ref, cnt_ref, w1_ref, b1_ref, g1_ref, be1_ref,
                 w2_ref, b2_ref, g2_ref, be2_ref, w3_ref, b3_ref, out_ref):
    cnt = jnp.maximum(cnt_ref[:, 0:1], 1.0)
    z = jnp.concatenate([sum_ref[...] / cnt, max_ref[...]], axis=1)

    def bn_exact(a, g, be):
        m = jnp.mean(a, axis=0, keepdims=True)
        v = jnp.mean((a - m) * (a - m), axis=0, keepdims=True)
        return jnp.maximum((a - m) / jnp.sqrt(v + 1e-5) * g + be, 0.0)

    a1 = bn_exact(_dot(z, w1_ref[...]) + b1_ref[...], g1_ref[...], be1_ref[...])
    a2 = bn_exact(_dot(a1, w2_ref[...]) + b2_ref[...], g2_ref[...], be2_ref[...])
    out_ref[...] = _dot(a2, w3_ref[...]) + b3_ref[...]


@jax.jit
def _tc_head(sums, maxs, cnt, p):
    return pl.pallas_call(
        _head_kernel,
        out_shape=jax.ShapeDtypeStruct((_G, 8), _f32),
    )(sums, maxs, cnt,
      p["lin1_W"], p["lin1_b"].reshape(1, _H),
      p["gf1"].reshape(1, _H), p["bf1"].reshape(1, _H),
      p["lin2_W"], p["lin2_b"].reshape(1, _HH),
      p["gf2"].reshape(1, _HH), p["bf2"].reshape(1, _HH),
      jnp.pad(p["lin3_W"], ((0, 0), (0, 6))),
      jnp.pad(p["lin3_b"], (0, 6)).reshape(1, 8))


def kernel(x, edge_index, batch, params):
    pad = _EP - _E
    idx = jnp.arange(pad, dtype=jnp.int32)
    src2 = jnp.concatenate(
        [edge_index[0].astype(jnp.int32), idx * 17 % _N]).reshape(_EROWS, 128)
    dst2 = jnp.concatenate(
        [edge_index[1].astype(jnp.int32),
         _N + idx % (_ACC_ROWS - _N)]).reshape(_EROWS, 128)
    x8 = jnp.pad(x, ((0, 0), (0, 5)))
    w1 = jnp.pad(params["W1"], ((0, 5), (0, 0)))
    batch2 = batch.astype(jnp.int32).reshape(_N, 1)

    deg_a, deg_b = _sc_degree(dst2)
    dinv8, lo, hi = _tc_first(deg_a, deg_b, x8, w1)

    for l, wn in (("1", "W2"), ("2", "W3"), ("3", None)):
        agg_lo, agg_hi = _sc_aggregate(src2, dst2, lo, hi)
        z, st = _tc_stats(agg_lo, agg_hi, lo, hi, dinv8,
                          params["b" + l].reshape(1, _H))
        gvec = params["g" + l].reshape(1, _H)
        bevec = params["be" + l].reshape(1, _H)
        if wn is not None:
            lo, hi = _tc_apply(z, st, gvec, bevec, params[wn], dinv8)
        else:
            h3 = _tc_final(z, st, gvec, bevec)

    sums, maxs, cnt = _tc_pool(h3, batch2)
    out = _tc_head(sums, maxs, cnt, params)
    return out[:, :2]
